# Initial kernel scaffold; baseline (speedup 1.0000x reference)
import jax
import jax.numpy as jnp
from jax.experimental import pallas as pl


def kernel(x, w_gate, W_experts):
    raise NotImplementedError("write your pallas kernel here")


# speedup vs baseline: 2.3303x; 2.3303x over previous
import functools

import jax
import jax.numpy as jnp
from jax.experimental import pallas as pl


def _routing_body(x_ref, wg_ref, gates_ref, imp_ref, load_ref, *, n_experts):
    xb = x_ref[...]
    wg = wg_ref[...]
    logits = jnp.dot(xb, wg, preferred_element_type=jnp.float32)
    tn = logits.shape[0]
    iota = jax.lax.broadcasted_iota(jnp.int32, (tn, n_experts), 1)

    m1 = jnp.max(logits, axis=1, keepdims=True)
    i1 = jnp.min(jnp.where(logits == m1, iota, n_experts), axis=1, keepdims=True)
    masked = jnp.where(iota == i1, -jnp.inf, logits)
    m2 = jnp.max(masked, axis=1, keepdims=True)
    i2 = jnp.min(jnp.where(masked == m2, iota, n_experts), axis=1, keepdims=True)

    e2 = jnp.exp(m2 - m1)
    denom = 1.0 + e2
    g1 = 1.0 / denom
    g2 = e2 / denom

    gates = jnp.where(iota == i1, g1, 0.0) + jnp.where(iota == i2, g2, 0.0)
    gates_ref[...] = gates
    imp_ref[0, 0, :] = jnp.sum(gates, axis=0)
    load_ref[0, 0, :] = jnp.sum((gates > 0.0).astype(jnp.float32), axis=0)


def _moe_body(x_ref, gates_ref, w_ref, out_ref, *, n_experts):
    e = pl.program_id(1)
    xb = x_ref[...]
    wb = w_ref[0]
    g = gates_ref[...]
    col = jax.lax.broadcasted_iota(jnp.int32, (1, n_experts), 1) == e
    gcol = jnp.sum(g * col.astype(g.dtype), axis=1, keepdims=True)
    acc = jnp.dot(xb * gcol, wb, preferred_element_type=jnp.float32)

    @pl.when(e == 0)
    def _init():
        out_ref[...] = acc

    @pl.when(e != 0)
    def _acc():
        out_ref[...] += acc


def _cv_squared(v):
    eps = 1e-10
    v = v.astype(jnp.float32)
    m = jnp.mean(v)
    var = jnp.var(v, ddof=1)
    return var / (m ** 2 + eps)


@jax.jit
def kernel(x, w_gate, W_experts):
    orig_shape = x.shape[:-1]
    D = x.shape[-1]
    E, _, H = W_experts.shape
    xf = x.reshape(-1, D)
    N = xf.shape[0]

    TN = 2048
    n_rblocks = N // TN
    gates, imp, load = pl.pallas_call(
        functools.partial(_routing_body, n_experts=E),
        grid=(n_rblocks,),
        in_specs=[
            pl.BlockSpec((TN, D), lambda t: (t, 0)),
            pl.BlockSpec((D, E), lambda t: (0, 0)),
        ],
        out_specs=[
            pl.BlockSpec((TN, E), lambda t: (t, 0)),
            pl.BlockSpec((1, 1, E), lambda t: (t, 0, 0)),
            pl.BlockSpec((1, 1, E), lambda t: (t, 0, 0)),
        ],
        out_shape=[
            jax.ShapeDtypeStruct((N, E), jnp.float32),
            jax.ShapeDtypeStruct((n_rblocks, 1, E), jnp.float32),
            jax.ShapeDtypeStruct((n_rblocks, 1, E), jnp.float32),
        ],
    )(xf, w_gate)

    importance = jnp.sum(imp.reshape(n_rblocks, E), axis=0)
    load_t = jnp.sum(load.reshape(n_rblocks, E), axis=0)
    loss = (_cv_squared(importance) + _cv_squared(load_t)) * 0.01

    TM = 1024
    n_mblocks = N // TM
    y = pl.pallas_call(
        functools.partial(_moe_body, n_experts=E),
        grid=(n_mblocks, E),
        in_specs=[
            pl.BlockSpec((TM, D), lambda t, e: (t, 0)),
            pl.BlockSpec((TM, E), lambda t, e: (t, 0)),
            pl.BlockSpec((1, D, H), lambda t, e: (e, 0, 0)),
        ],
        out_specs=pl.BlockSpec((TM, H), lambda t, e: (t, 0)),
        out_shape=jax.ShapeDtypeStruct((N, H), jnp.float32),
    )(xf, gates, W_experts)

    return (y.reshape(orig_shape + (H,)), loss)


# speedup vs baseline: 2.4806x; 1.0645x over previous
import functools

import jax
import jax.numpy as jnp
from jax.experimental import pallas as pl


def _routing_body(x_ref, wg_ref, gates_ref, imp_ref, load_ref, *, n_experts):
    xb = x_ref[...]
    wg = wg_ref[...]
    logits = jnp.dot(xb, wg, preferred_element_type=jnp.float32)
    tn = logits.shape[0]
    iota = jax.lax.broadcasted_iota(jnp.int32, (tn, n_experts), 1)

    m1 = jnp.max(logits, axis=1, keepdims=True)
    i1 = jnp.min(jnp.where(logits == m1, iota, n_experts), axis=1, keepdims=True)
    masked = jnp.where(iota == i1, -jnp.inf, logits)
    m2 = jnp.max(masked, axis=1, keepdims=True)
    i2 = jnp.min(jnp.where(masked == m2, iota, n_experts), axis=1, keepdims=True)

    e2 = jnp.exp(m2 - m1)
    denom = 1.0 + e2
    g1 = 1.0 / denom
    g2 = e2 / denom

    gates = jnp.where(iota == i1, g1, 0.0) + jnp.where(iota == i2, g2, 0.0)
    gates_ref[...] = gates
    imp_ref[0, 0, :] = jnp.sum(gates, axis=0)
    load_ref[0, 0, :] = jnp.sum((gates > 0.0).astype(jnp.float32), axis=0)


def _moe_body(x_ref, gates_ref, w_ref, out_ref, *, n_experts):
    e = pl.program_id(1)
    xb = x_ref[...]
    wb = w_ref[0]
    g = gates_ref[...]
    col = jax.lax.broadcasted_iota(jnp.int32, (1, n_experts), 1) == e
    gcol = jnp.sum(g * col.astype(g.dtype), axis=1, keepdims=True)
    acc = jnp.dot(xb * gcol, wb, preferred_element_type=jnp.float32)

    @pl.when(e == 0)
    def _init():
        out_ref[...] = acc

    @pl.when(e != 0)
    def _acc():
        out_ref[...] += acc


def _cv_squared(v):
    eps = 1e-10
    v = v.astype(jnp.float32)
    m = jnp.mean(v)
    var = jnp.var(v, ddof=1)
    return var / (m ** 2 + eps)


@jax.jit
def kernel(x, w_gate, W_experts):
    orig_shape = x.shape[:-1]
    D = x.shape[-1]
    E, _, H = W_experts.shape
    xf = x.reshape(-1, D)
    N = xf.shape[0]

    TN = 2048
    n_rblocks = N // TN
    gates, imp, load = pl.pallas_call(
        functools.partial(_routing_body, n_experts=E),
        grid=(n_rblocks,),
        in_specs=[
            pl.BlockSpec((TN, D), lambda t: (t, 0)),
            pl.BlockSpec((D, E), lambda t: (0, 0)),
        ],
        out_specs=[
            pl.BlockSpec((TN, E), lambda t: (t, 0)),
            pl.BlockSpec((1, 1, E), lambda t: (t, 0, 0)),
            pl.BlockSpec((1, 1, E), lambda t: (t, 0, 0)),
        ],
        out_shape=[
            jax.ShapeDtypeStruct((N, E), jnp.float32),
            jax.ShapeDtypeStruct((n_rblocks, 1, E), jnp.float32),
            jax.ShapeDtypeStruct((n_rblocks, 1, E), jnp.float32),
        ],
    )(xf, w_gate)

    importance = jnp.sum(imp.reshape(n_rblocks, E), axis=0)
    load_t = jnp.sum(load.reshape(n_rblocks, E), axis=0)
    loss = (_cv_squared(importance) + _cv_squared(load_t)) * 0.01

    TM = 2048
    n_mblocks = N // TM
    y = pl.pallas_call(
        functools.partial(_moe_body, n_experts=E),
        grid=(n_mblocks, E),
        in_specs=[
            pl.BlockSpec((TM, D), lambda t, e: (t, 0)),
            pl.BlockSpec((TM, E), lambda t, e: (t, 0)),
            pl.BlockSpec((1, D, H), lambda t, e: (e, 0, 0)),
        ],
        out_specs=pl.BlockSpec((TM, H), lambda t, e: (t, 0)),
        out_shape=jax.ShapeDtypeStruct((N, H), jnp.float32),
    )(xf, gates, W_experts)

    return (y.reshape(orig_shape + (H,)), loss)
